# baseline (speedup 1.0000x reference)
import jax
import jax.numpy as jnp
from jax import lax
from jax.experimental import pallas as pl
from jax.experimental.pallas import tpu as pltpu
from jax.experimental.pallas import tpu_sc as plsc

NC = 2
NS = 16

B, S, D = 4096, 200, 3
V = 1000000
N = B * S
NT = N // NS
W = 3200
TOT = NT // W
HALF = TOT // 2
NBUF = 4
VS = V // 8


def _stage_column(col_hbm, shA, sid):
    for t in range(8):
        @pl.when(sid == t)
        def _():
            pltpu.sync_copy(col_hbm.at[pl.ds(t * VS, VS)],
                            shA.at[pl.ds(t * VS, VS)])


def _body(tok_hbm, colA_hbm, colB_hbm, colC_hbm, out_hbm,
          ibufs, shA, bufs, flo, fhi, isems, gsems, wsems, wsems2):
    core = lax.axis_index("c")
    sid = lax.axis_index("s")

    @pl.when(core == 0)
    def _():
        _stage_column(colA_hbm, shA, sid)

    @pl.when(core == 1)
    def _():
        _stage_column(colC_hbm, shA, sid)
    plsc.subcore_barrier()

    def run(tasks):
        idd = [None] * len(tasks)
        gd = [None] * len(tasks)
        wlo = [None] * len(tasks)
        whi = [None] * len(tasks)

        def issue_gather(k):
            b = k % NBUF
            idd[k].wait()
            gd[k] = pltpu.async_copy(shA.at[ibufs[b]], bufs[b], gsems[b])

        def unpack_and_write(k):
            p, g = tasks[k]
            b = k % NBUF
            gd[k].wait()

            def cv(j, carry):
                sl = pl.ds(j * 16, 16)
                x = bufs[b][sl]
                flo[b][sl] = plsc.bitcast(x << jnp.uint32(16), jnp.float32)
                fhi[b][sl] = plsc.bitcast(
                    x & jnp.uint32(0xFFFF0000), jnp.float32)
                return carry
            lax.fori_loop(0, W // 16, cv, 0)
            off = (sid * TOT + g) * W
            wlo[k] = pltpu.async_copy(
                flo[b], out_hbm.at[2 * p, pl.ds(off, W)], wsems[b])
            whi[k] = pltpu.async_copy(
                fhi[b], out_hbm.at[2 * p + 1, pl.ds(off, W)], wsems2[b])

        for k, (p, g) in enumerate(tasks):
            b = k % NBUF
            if k >= NBUF:
                wlo[k - NBUF].wait()
                whi[k - NBUF].wait()
            idd[k] = pltpu.async_copy(
                tok_hbm.at[pl.ds((sid * TOT + g) * W, W)], ibufs[b], isems[b])
            if k >= 1:
                issue_gather(k - 1)
            if k >= 3:
                unpack_and_write(k - 3)
        n = len(tasks)
        if n >= 1:
            issue_gather(n - 1)
        for k in range(max(0, n - 3), n):
            unpack_and_write(k)
        for k in range(max(0, n - NBUF), n):
            wlo[k].wait()
            whi[k].wait()

    @pl.when(core == 0)
    def _():
        run([(0, g) for g in range(TOT)])

    @pl.when(core == 1)
    def _():
        run([(2, g) for g in range(TOT)])

    plsc.subcore_barrier()
    _stage_column(colB_hbm, shA, sid)
    plsc.subcore_barrier()

    @pl.when(core == 0)
    def _():
        run([(1, g) for g in range(HALF)])

    @pl.when(core == 1)
    def _():
        run([(1, g) for g in range(HALF, TOT)])


@jax.jit
def _wave_embed(tok2d, colA, colB, colC):
    mesh = plsc.VectorSubcoreMesh(
        core_axis_name="c", subcore_axis_name="s",
        num_cores=NC, num_subcores=NS)
    return pl.kernel(
        _body,
        out_type=jax.ShapeDtypeStruct((2 * D, N), jnp.float32),
        mesh=mesh,
        compiler_params=pltpu.CompilerParams(
            needs_layout_passes=False, use_tc_tiling_on_sc=False),
        scratch_types=[
            [pltpu.VMEM((W,), jnp.int32) for _ in range(NBUF)],
            pltpu.VMEM_SHARED((V,), jnp.uint32),
            [pltpu.VMEM((W,), jnp.uint32) for _ in range(NBUF)],
            [pltpu.VMEM((W,), jnp.float32) for _ in range(NBUF)],
            [pltpu.VMEM((W,), jnp.float32) for _ in range(NBUF)],
            [pltpu.SemaphoreType.DMA for _ in range(NBUF)],
            [pltpu.SemaphoreType.DMA for _ in range(NBUF)],
            [pltpu.SemaphoreType.DMA for _ in range(NBUF)],
            [pltpu.SemaphoreType.DMA for _ in range(NBUF)],
        ],
    )(tok2d, colA, colB, colC)


def kernel(token_ids, frequencies, amplitudes):
    tok = token_ids.T.reshape(-1).astype(jnp.int32)

    def rne_hi(x):
        return (x + jnp.uint32(0x7FFF) + ((x >> 16) & jnp.uint32(1))) \
            & jnp.uint32(0xFFFF0000)

    fbits = jax.lax.bitcast_convert_type(frequencies.T, jnp.uint32)
    abits = jax.lax.bitcast_convert_type(amplitudes.T, jnp.uint32)
    colA = (rne_hi(fbits[0]) >> 16) | rne_hi(fbits[1])
    colB = (rne_hi(fbits[2]) >> 16) | rne_hi(abits[0])
    colC = (rne_hi(abits[1]) >> 16) | rne_hi(abits[2])

    out6 = _wave_embed(tok, colA, colB, colC)

    return out6.reshape(2 * D, S, B).transpose(2, 1, 0)


# speedup vs baseline: 1.1425x; 1.1425x over previous
import jax
import jax.numpy as jnp
from jax import lax
from jax.experimental import pallas as pl
from jax.experimental.pallas import tpu as pltpu
from jax.experimental.pallas import tpu_sc as plsc

NC = 2
NS = 16

B, S, D = 4096, 200, 3
V = 1000000
N = B * S
NT = N // NS
W = 3200
TOT = NT // W
HALF = TOT // 2
NBUF = 4
VS = V // 8


def _stage_column(col_hbm, shA, sid):
    for t in range(8):
        @pl.when(sid == t)
        def _():
            pltpu.sync_copy(col_hbm.at[pl.ds(t * VS, VS)],
                            shA.at[pl.ds(t * VS, VS)])


def _body(tok_hbm, colA_hbm, colB_hbm, colC_hbm, out_hbm,
          ibufs, shA, bufs, flo, fhi, isems, gsems, wsems, wsems2):
    core = lax.axis_index("c")
    sid = lax.axis_index("s")

    @pl.when(core == 0)
    def _():
        _stage_column(colA_hbm, shA, sid)

    @pl.when(core == 1)
    def _():
        _stage_column(colC_hbm, shA, sid)
    plsc.subcore_barrier()

    def run(tasks):
        idd = [None] * len(tasks)
        gd = [None] * len(tasks)
        wlo = [None] * len(tasks)
        whi = [None] * len(tasks)

        def issue_gather(k):
            b = k % NBUF
            idd[k].wait()
            gd[k] = pltpu.async_copy(shA.at[ibufs[b]], bufs[b], gsems[b])

        def unpack_and_write(k):
            p, g = tasks[k]
            b = k % NBUF
            gd[k].wait()

            def cv(j, carry):
                sl = pl.ds(j * 16, 16)
                x = bufs[b][sl]
                flo[b][sl] = plsc.bitcast(x << jnp.uint32(16), jnp.float32)
                fhi[b][sl] = plsc.bitcast(
                    x & jnp.uint32(0xFFFF0000), jnp.float32)
                return carry
            lax.fori_loop(0, W // 16, cv, 0)
            off = (sid * TOT + g) * W
            wlo[k] = pltpu.async_copy(
                flo[b], out_hbm.at[2 * p, pl.ds(off, W)], wsems[b])
            whi[k] = pltpu.async_copy(
                fhi[b], out_hbm.at[2 * p + 1, pl.ds(off, W)], wsems2[b])

        for k, (p, g) in enumerate(tasks):
            b = k % NBUF
            if k >= NBUF:
                wlo[k - NBUF].wait()
                whi[k - NBUF].wait()
            idd[k] = pltpu.async_copy(
                tok_hbm.at[pl.ds((sid * TOT + g) * W, W)], ibufs[b], isems[b])
            if k >= 1:
                issue_gather(k - 1)
            if k >= 3:
                unpack_and_write(k - 3)
        n = len(tasks)
        if n >= 1:
            issue_gather(n - 1)
        for k in range(max(0, n - 3), n):
            unpack_and_write(k)
        for k in range(max(0, n - NBUF), n):
            wlo[k].wait()
            whi[k].wait()

    @pl.when(core == 0)
    def _():
        run([(0, g) for g in range(TOT)])

    @pl.when(core == 1)
    def _():
        run([(2, g) for g in range(TOT)])

    plsc.subcore_barrier()
    _stage_column(colB_hbm, shA, sid)
    plsc.subcore_barrier()

    @pl.when(core == 0)
    def _():
        run([(1, g) for g in range(HALF)])

    @pl.when(core == 1)
    def _():
        run([(1, g) for g in range(HALF, TOT)])


@jax.jit
def _wave_embed(tok2d, colA, colB, colC):
    mesh = plsc.VectorSubcoreMesh(
        core_axis_name="c", subcore_axis_name="s",
        num_cores=NC, num_subcores=NS)
    return pl.kernel(
        _body,
        out_type=jax.ShapeDtypeStruct((2 * D, N), jnp.float32),
        mesh=mesh,
        compiler_params=pltpu.CompilerParams(
            needs_layout_passes=False, use_tc_tiling_on_sc=False),
        scratch_types=[
            [pltpu.VMEM((W,), jnp.int32) for _ in range(NBUF)],
            pltpu.VMEM_SHARED((V,), jnp.uint32),
            [pltpu.VMEM((W,), jnp.uint32) for _ in range(NBUF)],
            [pltpu.VMEM((W,), jnp.float32) for _ in range(NBUF)],
            [pltpu.VMEM((W,), jnp.float32) for _ in range(NBUF)],
            [pltpu.SemaphoreType.DMA for _ in range(NBUF)],
            [pltpu.SemaphoreType.DMA for _ in range(NBUF)],
            [pltpu.SemaphoreType.DMA for _ in range(NBUF)],
            [pltpu.SemaphoreType.DMA for _ in range(NBUF)],
        ],
    )(tok2d, colA, colB, colC)


def kernel(token_ids, frequencies, amplitudes):
    tok = token_ids.T.reshape(-1).astype(jnp.int32)

    fu = jax.lax.bitcast_convert_type(
        frequencies.T.astype(jnp.bfloat16), jnp.uint16).astype(jnp.uint32)
    au = jax.lax.bitcast_convert_type(
        amplitudes.T.astype(jnp.bfloat16), jnp.uint16).astype(jnp.uint32)
    colA = fu[0] | (fu[1] << 16)
    colB = fu[2] | (au[0] << 16)
    colC = au[1] | (au[2] << 16)

    out6 = _wave_embed(tok, colA, colB, colC)

    return out6.reshape(2 * D, S, B).transpose(2, 1, 0)
